# Initial kernel scaffold; baseline (speedup 1.0000x reference)
import jax
import jax.numpy as jnp
from jax.experimental import pallas as pl


def kernel(x, edge_index, W1, b1, W2, b2):
    raise NotImplementedError("write your pallas kernel here")


# speedup vs baseline: 12.4634x; 12.4634x over previous
import functools

import jax
import jax.numpy as jnp
from jax import lax
from jax.experimental import pallas as pl
from jax.experimental.pallas import tpu as pltpu
from jax.experimental.pallas import tpu_sc as plsc

N = 10000
D = 128
E = 320000
NC = 2
NS = 16
NW = NC * NS
EPT = E // NW
CHUNK = 80
NCHUNK = EPT // CHUNK
FCH = 80
NFCH = N // FCH
FPS = -(-NFCH // NS)
DEGW = 16

_mesh = plsc.VectorSubcoreMesh(core_axis_name="c", subcore_axis_name="s")


def _deg_body(dst_hbm, out_hbm, idx_d, ones_v, zbuf, acc_sh):
    c = lax.axis_index("c")
    s = lax.axis_index("s")
    wid = s * NC + c

    zero16 = jnp.zeros((DEGW,), jnp.float32)
    one16 = jnp.ones((DEGW,), jnp.float32)

    @pl.loop(0, FCH)
    def _(r):
        zbuf[r, :] = zero16

    @pl.loop(0, CHUNK)
    def _(r):
        ones_v[r, :] = one16

    @pl.loop(0, FPS)
    def _(j):
        ch = s + j * NS

        @pl.when(ch < NFCH)
        def _():
            pltpu.sync_copy(zbuf, acc_sh.at[pl.ds(ch * FCH, FCH)])

    plsc.subcore_barrier()

    @pl.loop(0, NCHUNK)
    def _(i):
        base = wid * EPT + i * CHUNK
        pltpu.sync_copy(dst_hbm.at[pl.ds(base, CHUNK)], idx_d)
        pltpu.sync_copy(ones_v, acc_sh.at[idx_d], add=True)

    plsc.subcore_barrier()

    @pl.loop(0, FPS)
    def _(j):
        ch = s + j * NS

        @pl.when(ch < NFCH)
        def _():
            pltpu.sync_copy(acc_sh.at[pl.ds(ch * FCH, FCH)],
                            out_hbm.at[c, pl.ds(ch * FCH, FCH)])


_deg_call = pl.kernel(
    _deg_body,
    out_type=jax.ShapeDtypeStruct((NC, N, DEGW), jnp.float32),
    mesh=_mesh,
    scratch_types=[
        pltpu.VMEM((CHUNK,), jnp.int32),
        pltpu.VMEM((CHUNK, DEGW), jnp.float32),
        pltpu.VMEM((FCH, DEGW), jnp.float32),
        pltpu.VMEM_SHARED((N, DEGW), jnp.float32),
    ],
)


def _agg_body(src_hbm, dst_hbm, g_hbm, out_hbm,
              idx_s, idx_d, rows, zbuf, acc_sh, sem):
    c = lax.axis_index("c")
    s = lax.axis_index("s")
    wid = s * NC + c

    zero16 = jnp.zeros((16,), jnp.float32)

    @pl.loop(0, FCH)
    def _(r):
        @pl.loop(0, D // 16)
        def _(k):
            zbuf[r, pl.ds(k * 16, 16)] = zero16

    @pl.loop(0, FPS)
    def _(j):
        ch = s + j * NS

        @pl.when(ch < NFCH)
        def _():
            pltpu.sync_copy(zbuf, acc_sh.at[pl.ds(ch * FCH, FCH)])

    plsc.subcore_barrier()

    @pl.loop(0, NCHUNK)
    def _(i):
        base = wid * EPT + i * CHUNK
        pltpu.sync_copy(src_hbm.at[pl.ds(base, CHUNK)], idx_s)
        pltpu.sync_copy(dst_hbm.at[pl.ds(base, CHUNK)], idx_d)
        pltpu.async_copy(g_hbm.at[idx_s], rows, sem).wait()
        pltpu.sync_copy(rows, acc_sh.at[idx_d], add=True)

    plsc.subcore_barrier()

    @pl.loop(0, FPS)
    def _(j):
        ch = s + j * NS

        @pl.when(ch < NFCH)
        def _():
            pltpu.sync_copy(acc_sh.at[pl.ds(ch * FCH, FCH)],
                            out_hbm.at[c, pl.ds(ch * FCH, FCH)])


_agg_call = pl.kernel(
    _agg_body,
    out_type=jax.ShapeDtypeStruct((NC, N, D), jnp.float32),
    mesh=_mesh,
    scratch_types=[
        pltpu.VMEM((CHUNK,), jnp.int32),
        pltpu.VMEM((CHUNK,), jnp.int32),
        pltpu.VMEM((CHUNK, D), jnp.float32),
        pltpu.VMEM((FCH, D), jnp.float32),
        pltpu.VMEM_SHARED((N, D), jnp.float32),
        pltpu.SemaphoreType.DMA,
    ],
)


RB = 400


def _dinv_block(d0_ref, d1_ref):
    deg = d0_ref[:, 0:1] + d1_ref[:, 0:1] + 1.0
    return lax.rsqrt(deg)


def _tc1_body(x_ref, w_ref, d0_ref, d1_ref, g_ref):
    dinv = _dinv_block(d0_ref, d1_ref)
    h = jnp.dot(x_ref[...], w_ref[...], preferred_element_type=jnp.float32)
    g_ref[...] = h * dinv


def _tc2_body(a0_ref, a1_ref, g_ref, d0_ref, d1_ref, b_ref, w_ref, o_ref):
    dinv = _dinv_block(d0_ref, d1_ref)
    y = (a0_ref[...] + a1_ref[...] + g_ref[...]) * dinv + b_ref[...]
    y = jnp.maximum(y, 0.0)
    h = jnp.dot(y, w_ref[...], preferred_element_type=jnp.float32)
    o_ref[...] = h * dinv


def _tc3_body(a0_ref, a1_ref, g_ref, d0_ref, d1_ref, b_ref, o_ref):
    dinv = _dinv_block(d0_ref, d1_ref)
    z = (a0_ref[...] + a1_ref[...] + g_ref[...]) * dinv + b_ref[...]
    o = 1.0 / (1.0 + jnp.exp(-z))
    col = lax.broadcasted_iota(jnp.int32, (RB, D), 1)
    o_ref[...] = jnp.where(col >= D - 5, 1.0, o)


def _row_spec(width):
    return pl.BlockSpec((RB, width), lambda i: (i, 0))


def _full_spec(shape):
    return pl.BlockSpec(shape, lambda i: (0,) * len(shape))


_tc1_call = pl.pallas_call(
    _tc1_body,
    grid=(N // RB,),
    in_specs=[_row_spec(D), _full_spec((D, D)), _row_spec(DEGW), _row_spec(DEGW)],
    out_specs=_row_spec(D),
    out_shape=jax.ShapeDtypeStruct((N, D), jnp.float32),
)

_tc2_call = pl.pallas_call(
    _tc2_body,
    grid=(N // RB,),
    in_specs=[_row_spec(D), _row_spec(D), _row_spec(D), _row_spec(DEGW),
              _row_spec(DEGW), _full_spec((1, D)), _full_spec((D, D))],
    out_specs=_row_spec(D),
    out_shape=jax.ShapeDtypeStruct((N, D), jnp.float32),
)

_tc3_call = pl.pallas_call(
    _tc3_body,
    grid=(N // RB,),
    in_specs=[_row_spec(D), _row_spec(D), _row_spec(D), _row_spec(DEGW),
              _row_spec(DEGW), _full_spec((1, D))],
    out_specs=_row_spec(D),
    out_shape=jax.ShapeDtypeStruct((N, D), jnp.float32),
)


def kernel(x, edge_index, W1, b1, W2, b2):
    ei = edge_index.astype(jnp.int32)
    src = ei[0]
    dst = ei[1]

    deg = _deg_call(dst)
    d0, d1 = deg[0], deg[1]

    g1 = _tc1_call(x, W1, d0, d1)
    acc1 = _agg_call(src, dst, g1)
    g2 = _tc2_call(acc1[0], acc1[1], g1, d0, d1, b1.reshape(1, D), W2)
    acc2 = _agg_call(src, dst, g2)
    out = _tc3_call(acc2[0], acc2[1], g2, d0, d1, b2.reshape(1, D))
    return out
